# Initial kernel scaffold; baseline (speedup 1.0000x reference)
import functools

import jax
import jax.numpy as jnp
from jax import lax
from jax.experimental import pallas as pl
from jax.experimental.pallas import tpu as pltpu
from jax.experimental.pallas import tpu_sc as plsc

B, T, N, F_DYN, F_STA = 2, 12, 10000, 16, 16
H, G, E = 64, 64, 320000

NTILE = 16
NCORE = 2
EPAD = 327680
UNIT = 512
UNITS_PER_CORE_TILE = EPAD // (NCORE * NTILE * UNIT)
UNITS_PER_BATCH_TILE = EPAD // (NTILE * UNIT)
NROW = 10240
RPT = NROW // NTILE

_vmesh = plsc.VectorSubcoreMesh(core_axis_name="c", subcore_axis_name="s")


def _zero_vmem_2d(buf, rows, cols):
    zv = jnp.zeros((16,), jnp.float32)

    @pl.loop(0, rows)
    def _(r):
        @pl.loop(0, cols // 16)
        def _(j):
            buf[r, pl.ds(j * 16, 16)] = zv


def _zero_vmem_1d(buf, n):
    zv = jnp.zeros((16,), jnp.float32)

    @pl.loop(0, n // 16)
    def _(j):
        buf[pl.ds(j * 16, 16)] = zv


def _k0_counts(dst2d):
    @functools.partial(
        pl.kernel,
        mesh=_vmesh,
        out_type=jax.ShapeDtypeStruct((NCORE * NROW,), jnp.float32),
        scratch_types=[
            pltpu.VMEM((4, 128), jnp.int32),
            pltpu.VMEM((128,), jnp.float32),
            pltpu.VMEM((RPT,), jnp.float32),
            pltpu.VMEM_SHARED((NROW,), jnp.float32),
        ],
    )
    def k0(dst_hbm, out_hbm, didx, ones_v, zbuf, cnt_sh):
        c = lax.axis_index("c")
        s = lax.axis_index("s")
        ov = jnp.ones((16,), jnp.float32)

        @pl.loop(0, 8)
        def _(j):
            ones_v[pl.ds(j * 16, 16)] = ov

        _zero_vmem_1d(zbuf, RPT)
        pltpu.sync_copy(zbuf, cnt_sh.at[pl.ds(s * RPT, RPT)])
        plsc.subcore_barrier()

        row0 = (c * NTILE + s) * (UNITS_PER_CORE_TILE * 4)

        @pl.loop(0, UNITS_PER_CORE_TILE)
        def _(u):
            pltpu.sync_copy(dst_hbm.at[pl.ds(row0 + u * 4, 4)], didx)
            for j in range(4):
                pltpu.sync_copy(ones_v, cnt_sh.at[didx.at[j]], add=True)

        plsc.subcore_barrier()
        pltpu.sync_copy(
            cnt_sh.at[pl.ds(s * RPT, RPT)],
            out_hbm.at[pl.ds(c * NROW + s * RPT, RPT)],
        )

    return k0(dst2d)


def _k2_seg64(x0, x1, src2d, dst2d):
    @functools.partial(
        pl.kernel,
        mesh=_vmesh,
        out_type=jax.ShapeDtypeStruct((B * N, G), jnp.float32),
        scratch_types=[
            pltpu.VMEM((4, 128), jnp.int32),
            pltpu.VMEM((4, 128), jnp.int32),
            pltpu.VMEM((UNIT, G), jnp.float32),
            pltpu.VMEM((128, G), jnp.float32),
            pltpu.VMEM_SHARED((NROW, G), jnp.float32),
        ],
    )
    def k2(x0_hbm, x1_hbm, src_hbm, dst_hbm, out_hbm, sidx, didx, vals, zbuf,
           acc_sh):
        c = lax.axis_index("c")
        s = lax.axis_index("s")

        _zero_vmem_2d(zbuf, 128, G)

        @pl.loop(0, RPT // 128)
        def _(k):
            pltpu.sync_copy(zbuf, acc_sh.at[pl.ds(s * RPT + k * 128, 128)])

        plsc.subcore_barrier()

        row0 = s * (UNITS_PER_BATCH_TILE * 4)

        def tile_body(x_hbm):
            @pl.loop(0, UNITS_PER_BATCH_TILE)
            def _(u):
                r = row0 + u * 4
                pltpu.sync_copy(src_hbm.at[pl.ds(r, 4)], sidx)
                pltpu.sync_copy(dst_hbm.at[pl.ds(r, 4)], didx)
                for j in range(4):
                    pltpu.sync_copy(x_hbm.at[sidx.at[j]],
                                    vals.at[pl.ds(j * 128, 128)])
                for j in range(4):
                    pltpu.sync_copy(vals.at[pl.ds(j * 128, 128)],
                                    acc_sh.at[didx.at[j]], add=True)

        @pl.when(c == 0)
        def _():
            tile_body(x0_hbm)

        @pl.when(c == 1)
        def _():
            tile_body(x1_hbm)

        plsc.subcore_barrier()

        last = (NTILE - 1) * RPT

        @pl.when(s < NTILE - 1)
        def _():
            pltpu.sync_copy(acc_sh.at[pl.ds(s * RPT, RPT)],
                            out_hbm.at[pl.ds(c * N + s * RPT, RPT)])

        @pl.when(s == NTILE - 1)
        def _():
            pltpu.sync_copy(acc_sh.at[pl.ds(last, N - last)],
                            out_hbm.at[pl.ds(c * N + last, N - last)])

    return k2(x0, x1, src2d, dst2d)


def _k4_seg1(y0, y1, src2d, dst2d):
    @functools.partial(
        pl.kernel,
        mesh=_vmesh,
        out_type=jax.ShapeDtypeStruct((NCORE * NROW,), jnp.float32),
        scratch_types=[
            pltpu.VMEM((4, 128), jnp.int32),
            pltpu.VMEM((4, 128), jnp.int32),
            pltpu.VMEM((UNIT,), jnp.float32),
            pltpu.VMEM((RPT,), jnp.float32),
            pltpu.VMEM_SHARED((NROW,), jnp.float32),
        ],
    )
    def k4(y0_hbm, y1_hbm, src_hbm, dst_hbm, out_hbm, sidx, didx, vals, zbuf,
           acc_sh):
        c = lax.axis_index("c")
        s = lax.axis_index("s")

        _zero_vmem_1d(zbuf, RPT)
        pltpu.sync_copy(zbuf, acc_sh.at[pl.ds(s * RPT, RPT)])
        plsc.subcore_barrier()

        row0 = s * (UNITS_PER_BATCH_TILE * 4)

        def tile_body(y_hbm):
            @pl.loop(0, UNITS_PER_BATCH_TILE)
            def _(u):
                r = row0 + u * 4
                pltpu.sync_copy(src_hbm.at[pl.ds(r, 4)], sidx)
                pltpu.sync_copy(dst_hbm.at[pl.ds(r, 4)], didx)
                for j in range(4):
                    pltpu.sync_copy(y_hbm.at[sidx.at[j]],
                                    vals.at[pl.ds(j * 128, 128)])
                for j in range(4):
                    pltpu.sync_copy(vals.at[pl.ds(j * 128, 128)],
                                    acc_sh.at[didx.at[j]], add=True)

        @pl.when(c == 0)
        def _():
            tile_body(y0_hbm)

        @pl.when(c == 1)
        def _():
            tile_body(y1_hbm)

        plsc.subcore_barrier()
        pltpu.sync_copy(acc_sh.at[pl.ds(s * RPT, RPT)],
                        out_hbm.at[pl.ds(c * NROW + s * RPT, RPT)])

    return k4(y0, y1, src2d, dst2d)


def _k1_body(feat_ref, wih_ref, whh_ref, bih_ref, bhh_ref, ws_ref, bs_ref,
             wf_ref, bf_ref, out_ref):
    blk = feat_ref.shape[0]
    wih = wih_ref[...]
    whh = whh_ref[...]
    bias = bih_ref[...] + bhh_ref[...]
    h = jnp.zeros((blk, H), jnp.float32)
    c = jnp.zeros((blk, H), jnp.float32)
    dn = (((1,), (1,)), ((), ()))
    for t in range(T):
        x_t = feat_ref[:, t * F_DYN:(t + 1) * F_DYN]
        gates = (lax.dot_general(x_t, wih, dn, preferred_element_type=jnp.float32)
                 + lax.dot_general(h, whh, dn, preferred_element_type=jnp.float32)
                 + bias)
        gi = jax.nn.sigmoid(gates[:, 0 * H:1 * H])
        gf = jax.nn.sigmoid(gates[:, 1 * H:2 * H])
        gg = jnp.tanh(gates[:, 2 * H:3 * H])
        go = jax.nn.sigmoid(gates[:, 3 * H:4 * H])
        c = gf * c + gi * gg
        h = go * jnp.tanh(c)
    sta = feat_ref[:, T * F_DYN:T * F_DYN + F_STA]
    s_t = jax.nn.relu(
        lax.dot_general(sta, ws_ref[...], dn, preferred_element_type=jnp.float32)
        + bs_ref[...])
    wf = wf_ref[...]
    emb = jax.nn.relu(
        lax.dot_general(h, wf[:, :H], dn, preferred_element_type=jnp.float32)
        + lax.dot_general(s_t, wf[:, H:], dn, preferred_element_type=jnp.float32)
        + bf_ref[...])
    out_ref[...] = emb


def _k1_encode(feat, W_ih, W_hh, b_ih, b_hh, W_s, b_s, W_f, b_f):
    blk = 1000
    grid = (B * N // blk,)
    fw = T * F_DYN + F_STA
    return pl.pallas_call(
        _k1_body,
        grid=grid,
        in_specs=[
            pl.BlockSpec((blk, fw), lambda i: (i, 0)),
            pl.BlockSpec((4 * H, F_DYN), lambda i: (0, 0)),
            pl.BlockSpec((4 * H, H), lambda i: (0, 0)),
            pl.BlockSpec((1, 4 * H), lambda i: (0, 0)),
            pl.BlockSpec((1, 4 * H), lambda i: (0, 0)),
            pl.BlockSpec((H, F_STA), lambda i: (0, 0)),
            pl.BlockSpec((1, H), lambda i: (0, 0)),
            pl.BlockSpec((H, 2 * H), lambda i: (0, 0)),
            pl.BlockSpec((1, H), lambda i: (0, 0)),
        ],
        out_specs=pl.BlockSpec((blk, H), lambda i: (i, 0)),
        out_shape=jax.ShapeDtypeStruct((B * N, H), jnp.float32),
    )(feat, W_ih, W_hh, b_ih, b_hh, W_s, b_s, W_f, b_f)


def _k3_body(sum1_ref, x_ref, cnt_ref, wl1_ref, bl1_ref, wr1_ref, wl2_ref,
             wr2_ref, wo_ref, y_ref, z_ref):
    cnt = cnt_ref[0, :] + cnt_ref[1, :]
    inv = 1.0 / jnp.maximum(cnt, 1.0)
    mean = sum1_ref[...] * inv[:, None]
    dn = (((1,), (1,)), ((), ()))
    x1 = jax.nn.relu(
        lax.dot_general(mean, wl1_ref[...], dn, preferred_element_type=jnp.float32)
        + lax.dot_general(x_ref[...], wr1_ref[...], dn,
                          preferred_element_type=jnp.float32)
        + bl1_ref[...])
    wo = wo_ref[...]
    vl = lax.dot_general(wo, wl2_ref[...], (((1,), (0,)), ((), ())),
                         preferred_element_type=jnp.float32)
    vr = lax.dot_general(wo, wr2_ref[...], (((1,), (0,)), ((), ())),
                         preferred_element_type=jnp.float32)
    y_ref[0, :] = jnp.sum(x1 * vl, axis=1)
    z_ref[0, :] = jnp.sum(x1 * vr, axis=1)


def _k3_sage1(sum1, embed, cnt_parts, W_l1, b_l1, W_r1, W_l2, W_r2, W_o):
    blk = 2000
    nb = N // blk
    grid = (B * N // blk,)
    return pl.pallas_call(
        _k3_body,
        grid=grid,
        in_specs=[
            pl.BlockSpec((blk, G), lambda i: (i, 0)),
            pl.BlockSpec((blk, G), lambda i: (i, 0)),
            pl.BlockSpec((2, blk), lambda i: (0, i % nb)),
            pl.BlockSpec((G, G), lambda i: (0, 0)),
            pl.BlockSpec((1, G), lambda i: (0, 0)),
            pl.BlockSpec((G, G), lambda i: (0, 0)),
            pl.BlockSpec((G, G), lambda i: (0, 0)),
            pl.BlockSpec((G, G), lambda i: (0, 0)),
            pl.BlockSpec((1, G), lambda i: (0, 0)),
        ],
        out_specs=[
            pl.BlockSpec((1, blk), lambda i: (i // nb, i % nb)),
            pl.BlockSpec((1, blk), lambda i: (i // nb, i % nb)),
        ],
        out_shape=[
            jax.ShapeDtypeStruct((B, N), jnp.float32),
            jax.ShapeDtypeStruct((B, N), jnp.float32),
        ],
    )(sum1, embed, cnt_parts, W_l1, b_l1, W_r1, W_l2, W_r2, W_o)


def _k5_body(s2_ref, cnt_ref, z_ref, bl2_ref, wo_ref, bo_ref, out_ref):
    cnt = cnt_ref[0, :N] + cnt_ref[1, :N]
    inv = 1.0 / jnp.maximum(cnt, 1.0)
    c0 = jnp.sum(bl2_ref[...] * wo_ref[...]) + bo_ref[0, 0]
    out_ref[...] = s2_ref[:, :N] * inv[None, :] + z_ref[...] + c0


def _k5_combine(s2, cnt_parts, z, b_l2, W_o, b_o):
    return pl.pallas_call(
        _k5_body,
        grid=(1,),
        in_specs=[
            pl.BlockSpec((B, NROW), lambda i: (0, 0)),
            pl.BlockSpec((2, NROW), lambda i: (0, 0)),
            pl.BlockSpec((B, N), lambda i: (0, 0)),
            pl.BlockSpec((1, G), lambda i: (0, 0)),
            pl.BlockSpec((1, G), lambda i: (0, 0)),
            pl.BlockSpec((1, 1), lambda i: (0, 0)),
        ],
        out_specs=pl.BlockSpec((B, N), lambda i: (0, 0)),
        out_shape=jax.ShapeDtypeStruct((B, N), jnp.float32),
    )(s2, cnt_parts, z, b_l2, W_o, b_o)


def kernel(dynamic_features, static_features, edge_index, W_ih, W_hh, b_ih,
           b_hh, W_s, b_s, W_f, b_f, W_l1, b_l1, W_r1, W_l2, b_l2, W_r2, W_o,
           b_o):
    dyn = jnp.transpose(dynamic_features, (0, 2, 1, 3)).reshape(B * N, T * F_DYN)
    sta = static_features.reshape(B * N, F_STA)
    feat = jnp.concatenate([dyn, sta], axis=1)

    src = edge_index[0]
    dst = edge_index[1]
    npad = EPAD - E
    pad_src = (jnp.arange(npad, dtype=jnp.int32) * 37) % N
    pad_dst = N + (jnp.arange(npad, dtype=jnp.int32) % (NROW - N))
    src2d = jnp.concatenate([src, pad_src]).reshape(EPAD // 128, 128)
    dst2d = jnp.concatenate([dst, pad_dst]).reshape(EPAD // 128, 128)

    cnt_parts = _k0_counts(dst2d).reshape(2, NROW)

    embed = _k1_encode(feat, W_ih, W_hh, b_ih.reshape(1, 4 * H),
                       b_hh.reshape(1, 4 * H), W_s, b_s.reshape(1, H), W_f,
                       b_f.reshape(1, H))

    sum1 = _k2_seg64(embed[:N], embed[N:], src2d, dst2d)

    y1, z1 = _k3_sage1(sum1, embed, cnt_parts, W_l1, b_l1.reshape(1, G), W_r1,
                       W_l2, W_r2, W_o)

    s2 = _k4_seg1(y1[0], y1[1], src2d, dst2d).reshape(2, NROW)

    pred = _k5_combine(s2, cnt_parts, z1, b_l2.reshape(1, G), W_o,
                       b_o.reshape(1, 1))
    return pred


# speedup vs baseline: 11.2902x; 11.2902x over previous
import functools

import jax
import jax.numpy as jnp
from jax import lax
from jax.experimental import pallas as pl
from jax.experimental.pallas import tpu as pltpu
from jax.experimental.pallas import tpu_sc as plsc

B, T, N, F_DYN, F_STA = 2, 12, 10000, 16, 16
H, G, E = 64, 64, 320000

NTILE = 16
NCORE = 2
EPAD = 327680
UNIT = 512
UPT = EPAD // (NTILE * UNIT)
NP = 10240
RPT = NP // NTILE
GP = 64
FW = T * F_DYN + F_STA


def _vmesh():
    return plsc.VectorSubcoreMesh(core_axis_name="c", subcore_axis_name="s")


def _zero_vmem_2d(buf, rows, cols):
    zv = jnp.zeros((16,), jnp.float32)

    @pl.loop(0, rows)
    def _(r):
        @pl.loop(0, cols // 16)
        def _(j):
            buf[r, pl.ds(j * 16, 16)] = zv


def _zero_vmem_1d(buf, n):
    zv = jnp.zeros((16,), jnp.float32)

    @pl.loop(0, n // 16)
    def _(j):
        buf[pl.ds(j * 16, 16)] = zv


def _k0_counts(dst2d):
    @functools.partial(
        pl.kernel,
        mesh=_vmesh(),
        compiler_params=pltpu.CompilerParams(use_tc_tiling_on_sc=False),
        out_type=jax.ShapeDtypeStruct((NP,), jnp.float32),
        scratch_types=[
            pltpu.VMEM((4, 128), jnp.int32),
            pltpu.VMEM((128,), jnp.float32),
            pltpu.VMEM((RPT,), jnp.float32),
            pltpu.VMEM_SHARED((NP,), jnp.float32),
        ],
    )
    def k0(dst_hbm, out_hbm, didx, ones_v, zbuf, cnt_sh):
        c = lax.axis_index("c")
        s = lax.axis_index("s")

        @pl.when(c == 0)
        def _():
            ov = jnp.ones((16,), jnp.float32)

            @pl.loop(0, 8)
            def _(j):
                ones_v[pl.ds(j * 16, 16)] = ov

            _zero_vmem_1d(zbuf, RPT)
            pltpu.sync_copy(zbuf, cnt_sh.at[pl.ds(s * RPT, RPT)])
            plsc.subcore_barrier()

            row0 = s * (UPT * 4)

            @pl.loop(0, UPT)
            def _(u):
                pltpu.sync_copy(dst_hbm.at[pl.ds(row0 + u * 4, 4)], didx)
                for j in range(4):
                    pltpu.sync_copy(ones_v, cnt_sh.at[didx.at[j]], add=True)

            plsc.subcore_barrier()
            pltpu.sync_copy(cnt_sh.at[pl.ds(s * RPT, RPT)],
                            out_hbm.at[pl.ds(s * RPT, RPT)])

    return k0(dst2d)


def _k2_seg64(x0, x1, src2d, dst2d):
    @functools.partial(
        pl.kernel,
        mesh=_vmesh(),
        compiler_params=pltpu.CompilerParams(use_tc_tiling_on_sc=False),
        out_type=jax.ShapeDtypeStruct((B * NP, GP), jnp.float32),
        scratch_types=[
            pltpu.VMEM((4, 128), jnp.int32),
            pltpu.VMEM((4, 128), jnp.int32),
            pltpu.VMEM((UNIT, GP), jnp.float32),
            pltpu.VMEM((128, GP), jnp.float32),
            pltpu.VMEM_SHARED((NP, GP), jnp.float32),
        ],
    )
    def k2(x0_hbm, x1_hbm, src_hbm, dst_hbm, out_hbm, sidx, didx, vals, zbuf,
           acc_sh):
        c = lax.axis_index("c")
        s = lax.axis_index("s")

        _zero_vmem_2d(zbuf, 128, GP)

        @pl.loop(0, RPT // 128)
        def _(k):
            pltpu.sync_copy(zbuf, acc_sh.at[pl.ds(s * RPT + k * 128, 128)])

        plsc.subcore_barrier()

        row0 = s * (UPT * 4)

        def tile_body(x_hbm):
            @pl.loop(0, UPT)
            def _(u):
                r = row0 + u * 4
                pltpu.sync_copy(src_hbm.at[pl.ds(r, 4)], sidx)
                pltpu.sync_copy(dst_hbm.at[pl.ds(r, 4)], didx)
                for j in range(4):
                    pltpu.sync_copy(x_hbm.at[sidx.at[j]],
                                    vals.at[pl.ds(j * 128, 128)])
                for j in range(4):
                    pltpu.sync_copy(vals.at[pl.ds(j * 128, 128)],
                                    acc_sh.at[didx.at[j]], add=True)

        @pl.when(c == 0)
        def _():
            tile_body(x0_hbm)

        @pl.when(c == 1)
        def _():
            tile_body(x1_hbm)

        plsc.subcore_barrier()
        pltpu.sync_copy(acc_sh.at[pl.ds(s * RPT, RPT)],
                        out_hbm.at[pl.ds(c * NP + s * RPT, RPT)])

    return k2(x0, x1, src2d, dst2d)


def _k4_seg1(y0, y1, src2d, dst2d):
    @functools.partial(
        pl.kernel,
        mesh=_vmesh(),
        compiler_params=pltpu.CompilerParams(use_tc_tiling_on_sc=False),
        out_type=jax.ShapeDtypeStruct((B * NP,), jnp.float32),
        scratch_types=[
            pltpu.VMEM((4, 128), jnp.int32),
            pltpu.VMEM((4, 128), jnp.int32),
            pltpu.VMEM((UNIT,), jnp.float32),
            pltpu.VMEM((RPT,), jnp.float32),
            pltpu.VMEM_SHARED((NP,), jnp.float32),
        ],
    )
    def k4(y0_hbm, y1_hbm, src_hbm, dst_hbm, out_hbm, sidx, didx, vals, zbuf,
           acc_sh):
        c = lax.axis_index("c")
        s = lax.axis_index("s")

        _zero_vmem_1d(zbuf, RPT)
        pltpu.sync_copy(zbuf, acc_sh.at[pl.ds(s * RPT, RPT)])
        plsc.subcore_barrier()

        row0 = s * (UPT * 4)

        def tile_body(y_hbm):
            @pl.loop(0, UPT)
            def _(u):
                r = row0 + u * 4
                pltpu.sync_copy(src_hbm.at[pl.ds(r, 4)], sidx)
                pltpu.sync_copy(dst_hbm.at[pl.ds(r, 4)], didx)
                for j in range(4):
                    pltpu.sync_copy(y_hbm.at[sidx.at[j]],
                                    vals.at[pl.ds(j * 128, 128)])
                for j in range(4):
                    pltpu.sync_copy(vals.at[pl.ds(j * 128, 128)],
                                    acc_sh.at[didx.at[j]], add=True)

        @pl.when(c == 0)
        def _():
            tile_body(y0_hbm)

        @pl.when(c == 1)
        def _():
            tile_body(y1_hbm)

        plsc.subcore_barrier()
        pltpu.sync_copy(acc_sh.at[pl.ds(s * RPT, RPT)],
                        out_hbm.at[pl.ds(c * NP + s * RPT, RPT)])

    return k4(y0, y1, src2d, dst2d)


def _k1_body(feat_ref, wih_ref, whh_ref, bih_ref, bhh_ref, ws_ref, bs_ref,
             wf_ref, bf_ref, out_ref):
    blk = feat_ref.shape[0]
    wih = wih_ref[...]
    whh = whh_ref[...]
    bias = bih_ref[...] + bhh_ref[...]
    h = jnp.zeros((blk, H), jnp.float32)
    c = jnp.zeros((blk, H), jnp.float32)
    dn = (((1,), (1,)), ((), ()))
    for t in range(T):
        x_t = feat_ref[:, t * F_DYN:(t + 1) * F_DYN]
        gates = (lax.dot_general(x_t, wih, dn, preferred_element_type=jnp.float32)
                 + lax.dot_general(h, whh, dn, preferred_element_type=jnp.float32)
                 + bias)
        gi = jax.nn.sigmoid(gates[:, 0 * H:1 * H])
        gf = jax.nn.sigmoid(gates[:, 1 * H:2 * H])
        gg = jnp.tanh(gates[:, 2 * H:3 * H])
        go = jax.nn.sigmoid(gates[:, 3 * H:4 * H])
        c = gf * c + gi * gg
        h = go * jnp.tanh(c)
    sta = feat_ref[:, T * F_DYN:T * F_DYN + F_STA]
    s_t = jax.nn.relu(
        lax.dot_general(sta, ws_ref[...], dn, preferred_element_type=jnp.float32)
        + bs_ref[...])
    wf = wf_ref[...]
    emb = jax.nn.relu(
        lax.dot_general(h, wf[:, :H], dn, preferred_element_type=jnp.float32)
        + lax.dot_general(s_t, wf[:, H:], dn, preferred_element_type=jnp.float32)
        + bf_ref[...])
    if GP > H:
        emb = jnp.concatenate([emb, jnp.zeros((blk, GP - H), jnp.float32)],
                              axis=1)
    out_ref[...] = emb


def _k1_encode(feat, W_ih, W_hh, b_ih, b_hh, W_s, b_s, W_f, b_f):
    blk = 1024
    grid = (B * NP // blk,)
    return pl.pallas_call(
        _k1_body,
        grid=grid,
        in_specs=[
            pl.BlockSpec((blk, FW), lambda i: (i, 0)),
            pl.BlockSpec((4 * H, F_DYN), lambda i: (0, 0)),
            pl.BlockSpec((4 * H, H), lambda i: (0, 0)),
            pl.BlockSpec((1, 4 * H), lambda i: (0, 0)),
            pl.BlockSpec((1, 4 * H), lambda i: (0, 0)),
            pl.BlockSpec((H, F_STA), lambda i: (0, 0)),
            pl.BlockSpec((1, H), lambda i: (0, 0)),
            pl.BlockSpec((H, 2 * H), lambda i: (0, 0)),
            pl.BlockSpec((1, H), lambda i: (0, 0)),
        ],
        out_specs=pl.BlockSpec((blk, GP), lambda i: (i, 0)),
        out_shape=jax.ShapeDtypeStruct((B * NP, GP), jnp.float32),
    )(feat, W_ih, W_hh, b_ih, b_hh, W_s, b_s, W_f, b_f)


def _k3_body(sum1_ref, x_ref, cnt_ref, wl1_ref, bl1_ref, wr1_ref, wl2_ref,
             wr2_ref, wo_ref, y_ref, z_ref):
    inv = 1.0 / jnp.maximum(cnt_ref[...], 1.0)
    mean = sum1_ref[:, :G] * inv
    dn = (((1,), (1,)), ((), ()))
    x1 = jax.nn.relu(
        lax.dot_general(mean, wl1_ref[...], dn, preferred_element_type=jnp.float32)
        + lax.dot_general(x_ref[:, :G], wr1_ref[...], dn,
                          preferred_element_type=jnp.float32)
        + bl1_ref[...])
    wo = wo_ref[...]
    vl = lax.dot_general(wo, wl2_ref[...], (((1,), (0,)), ((), ())),
                         preferred_element_type=jnp.float32)
    vr = lax.dot_general(wo, wr2_ref[...], (((1,), (0,)), ((), ())),
                         preferred_element_type=jnp.float32)
    y_ref[...] = jnp.sum(x1 * vl, axis=1, keepdims=True)
    z_ref[...] = jnp.sum(x1 * vr, axis=1, keepdims=True)


def _k3_sage1(sum1, embed, cnt_col, W_l1, b_l1, W_r1, W_l2, W_r2, W_o):
    blk = 2048
    grid = (B * NP // blk,)
    return pl.pallas_call(
        _k3_body,
        grid=grid,
        in_specs=[
            pl.BlockSpec((blk, GP), lambda i: (i, 0)),
            pl.BlockSpec((blk, GP), lambda i: (i, 0)),
            pl.BlockSpec((blk, 1), lambda i: (i, 0)),
            pl.BlockSpec((G, G), lambda i: (0, 0)),
            pl.BlockSpec((1, G), lambda i: (0, 0)),
            pl.BlockSpec((G, G), lambda i: (0, 0)),
            pl.BlockSpec((G, G), lambda i: (0, 0)),
            pl.BlockSpec((G, G), lambda i: (0, 0)),
            pl.BlockSpec((1, G), lambda i: (0, 0)),
        ],
        out_specs=[
            pl.BlockSpec((blk, 1), lambda i: (i, 0)),
            pl.BlockSpec((blk, 1), lambda i: (i, 0)),
        ],
        out_shape=[
            jax.ShapeDtypeStruct((B * NP, 1), jnp.float32),
            jax.ShapeDtypeStruct((B * NP, 1), jnp.float32),
        ],
    )(sum1, embed, cnt_col, W_l1, b_l1, W_r1, W_l2, W_r2, W_o)


def _k5_body(s2_ref, cnt_ref, z_ref, bl2_ref, wo_ref, bo_ref, out_ref):
    cnt = cnt_ref[0, :N]
    inv = 1.0 / jnp.maximum(cnt, 1.0)
    c0 = jnp.sum(bl2_ref[...] * wo_ref[...]) + bo_ref[0, 0]
    out_ref[...] = s2_ref[:, :N] * inv[None, :] + z_ref[...] + c0


def _k5_combine(s2, cnt, z, b_l2, W_o, b_o):
    return pl.pallas_call(
        _k5_body,
        grid=(1,),
        in_specs=[
            pl.BlockSpec((B, NP), lambda i: (0, 0)),
            pl.BlockSpec((1, NP), lambda i: (0, 0)),
            pl.BlockSpec((B, N), lambda i: (0, 0)),
            pl.BlockSpec((1, G), lambda i: (0, 0)),
            pl.BlockSpec((1, G), lambda i: (0, 0)),
            pl.BlockSpec((1, 1), lambda i: (0, 0)),
        ],
        out_specs=pl.BlockSpec((B, N), lambda i: (0, 0)),
        out_shape=jax.ShapeDtypeStruct((B, N), jnp.float32),
    )(s2, cnt, z, b_l2, W_o, b_o)


def kernel(dynamic_features, static_features, edge_index, W_ih, W_hh, b_ih,
           b_hh, W_s, b_s, W_f, b_f, W_l1, b_l1, W_r1, W_l2, b_l2, W_r2, W_o,
           b_o):
    dyn = jnp.transpose(dynamic_features, (0, 2, 1, 3)).reshape(B, N, T * F_DYN)
    sta = static_features
    feat = jnp.concatenate([dyn, sta], axis=2)
    feat = jnp.pad(feat, ((0, 0), (0, NP - N), (0, 0))).reshape(B * NP, FW)

    src = edge_index[0]
    dst = edge_index[1]
    npad = EPAD - E
    pad_src = (jnp.arange(npad, dtype=jnp.int32) * 37) % N
    pad_dst = N + (jnp.arange(npad, dtype=jnp.int32) % (NP - N))
    src2d = jnp.concatenate([src, pad_src]).reshape(EPAD // 128, 128)
    dst2d = jnp.concatenate([dst, pad_dst]).reshape(EPAD // 128, 128)

    cnt = _k0_counts(dst2d)
    cnt_col = jnp.concatenate([cnt, cnt]).reshape(B * NP, 1)

    embed = _k1_encode(feat, W_ih, W_hh, b_ih.reshape(1, 4 * H),
                       b_hh.reshape(1, 4 * H), W_s, b_s.reshape(1, H), W_f,
                       b_f.reshape(1, H))

    sum1 = _k2_seg64(embed[:NP], embed[NP:], src2d, dst2d)

    y1, z1 = _k3_sage1(sum1, embed, cnt_col, W_l1, b_l1.reshape(1, G), W_r1,
                       W_l2, W_r2, W_o)

    yf = y1.reshape(B * NP)
    s2 = _k4_seg1(yf[:NP], yf[NP:], src2d, dst2d).reshape(B, NP)

    z = z1.reshape(B, NP)[:, :N]
    pred = _k5_combine(s2, cnt.reshape(1, NP), z, b_l2.reshape(1, G), W_o,
                       b_o.reshape(1, 1))
    return pred


# speedup vs baseline: 17.2711x; 1.5297x over previous
import functools

import jax
import jax.numpy as jnp
from jax import lax
from jax.experimental import pallas as pl
from jax.experimental.pallas import tpu as pltpu
from jax.experimental.pallas import tpu_sc as plsc

B, T, N, F_DYN, F_STA = 2, 12, 10000, 16, 16
H, G, E = 64, 64, 320000

NTILE = 16
NCORE = 2
EPAD = 327680
UNIT = 512
UPT = EPAD // (NTILE * UNIT)
NP = 10240
RPT = NP // NTILE
GP = 64
FW = T * F_DYN + F_STA


def _vmesh():
    return plsc.VectorSubcoreMesh(core_axis_name="c", subcore_axis_name="s")


def _zero_vmem_2d(buf, rows, cols):
    zv = jnp.zeros((16,), jnp.float32)

    @pl.loop(0, rows)
    def _(r):
        @pl.loop(0, cols // 16)
        def _(j):
            buf[r, pl.ds(j * 16, 16)] = zv


def _zero_vmem_1d(buf, n):
    zv = jnp.zeros((16,), jnp.float32)

    @pl.loop(0, n // 16)
    def _(j):
        buf[pl.ds(j * 16, 16)] = zv


def _k0_counts(dst2d):
    @functools.partial(
        pl.kernel,
        mesh=_vmesh(),
        compiler_params=pltpu.CompilerParams(use_tc_tiling_on_sc=False),
        out_type=jax.ShapeDtypeStruct((NP,), jnp.float32),
        scratch_types=[
            pltpu.VMEM((4, 128), jnp.int32),
            pltpu.VMEM((128,), jnp.float32),
            pltpu.VMEM((RPT,), jnp.float32),
            pltpu.VMEM_SHARED((NP,), jnp.float32),
        ],
    )
    def k0(dst_hbm, out_hbm, didx, ones_v, zbuf, cnt_sh):
        c = lax.axis_index("c")
        s = lax.axis_index("s")

        @pl.when(c == 0)
        def _():
            ov = jnp.ones((16,), jnp.float32)

            @pl.loop(0, 8)
            def _(j):
                ones_v[pl.ds(j * 16, 16)] = ov

            _zero_vmem_1d(zbuf, RPT)
            pltpu.sync_copy(zbuf, cnt_sh.at[pl.ds(s * RPT, RPT)])
            plsc.subcore_barrier()

            row0 = s * (UPT * 4)

            @pl.loop(0, UPT)
            def _(u):
                pltpu.sync_copy(dst_hbm.at[pl.ds(row0 + u * 4, 4)], didx)
                for j in range(4):
                    pltpu.sync_copy(ones_v, cnt_sh.at[didx.at[j]], add=True)

            plsc.subcore_barrier()
            pltpu.sync_copy(cnt_sh.at[pl.ds(s * RPT, RPT)],
                            out_hbm.at[pl.ds(s * RPT, RPT)])

    return k0(dst2d)


def _k2_seg64(x, src3, dst2d):
    @functools.partial(
        pl.kernel,
        mesh=_vmesh(),
        compiler_params=pltpu.CompilerParams(use_tc_tiling_on_sc=False),
        out_type=jax.ShapeDtypeStruct((B * NP, GP), jnp.float32),
        scratch_types=[
            pltpu.VMEM((2, 4, 128), jnp.int32),
            pltpu.VMEM((2, 4, 128), jnp.int32),
            pltpu.VMEM((2, UNIT, GP), jnp.float32),
            pltpu.VMEM((128, GP), jnp.float32),
            pltpu.VMEM_SHARED((NP, GP), jnp.float32),
            pltpu.SemaphoreType.DMA,
            pltpu.SemaphoreType.DMA,
            pltpu.SemaphoreType.DMA,
            pltpu.SemaphoreType.DMA,
            pltpu.SemaphoreType.DMA,
        ],
    )
    def k2(x_hbm, src_hbm, dst_hbm, out_hbm, sidx, didx, vals, zbuf, acc_sh,
           semi, semg0, semg1, sems0, sems1):
        c = lax.axis_index("c")
        s = lax.axis_index("s")
        semg = (semg0, semg1)
        sems = (sems0, sems1)

        _zero_vmem_2d(zbuf, 128, GP)

        @pl.loop(0, RPT // 128)
        def _(k):
            pltpu.sync_copy(zbuf, acc_sh.at[pl.ds(s * RPT + k * 128, 128)])

        plsc.subcore_barrier()

        row0 = s * (UPT * 4)

        def load_idx_sync(slot, u):
            r = row0 + u * 4
            h1 = pltpu.async_copy(src_hbm.at[c, pl.ds(r, 4)], sidx.at[slot],
                                  semi)
            h2 = pltpu.async_copy(dst_hbm.at[pl.ds(r, 4)], didx.at[slot], semi)
            h1.wait()
            h2.wait()

        def fire_g(slot):
            for j in range(4):
                pltpu.async_copy(x_hbm.at[sidx.at[slot, j]],
                                 vals.at[slot, pl.ds(j * 128, 128)],
                                 semg[slot])

        def wait_g(slot):
            for j in range(4):
                pltpu.make_async_copy(x_hbm.at[pl.ds(0, 128)],
                                      vals.at[slot, pl.ds(j * 128, 128)],
                                      semg[slot]).wait()

        def fire_s(slot):
            for j in range(4):
                pltpu.async_copy(vals.at[slot, pl.ds(j * 128, 128)],
                                 acc_sh.at[didx.at[slot, j]], sems[slot],
                                 add=True)

        def wait_s(slot):
            for j in range(4):
                pltpu.make_async_copy(vals.at[slot, pl.ds(j * 128, 128)],
                                      acc_sh.at[pl.ds(0, 128)],
                                      sems[slot]).wait()

        for slot in (0, 1):
            load_idx_sync(slot, slot)
            fire_g(slot)
            wait_g(slot)
            fire_s(slot)

        @pl.loop(0, (UPT - 2) // 2)
        def _(p):
            u = 2 + p * 2
            for slot in (0, 1):
                wait_s(slot)
                load_idx_sync(slot, u + slot)
                fire_g(slot)
                wait_g(slot)
                fire_s(slot)

        wait_s(0)
        wait_s(1)

        plsc.subcore_barrier()
        pltpu.sync_copy(acc_sh.at[pl.ds(s * RPT, RPT)],
                        out_hbm.at[pl.ds(c * NP + s * RPT, RPT)])

    return k2(x, src3, dst2d)


U4 = 2048
UPT4 = EPAD // (NTILE * U4)


def _k4_seg1(y, src3, dst2d):
    @functools.partial(
        pl.kernel,
        mesh=_vmesh(),
        compiler_params=pltpu.CompilerParams(use_tc_tiling_on_sc=False),
        out_type=jax.ShapeDtypeStruct((B * NP,), jnp.float32),
        scratch_types=[
            pltpu.VMEM((2, 16, 128), jnp.int32),
            pltpu.VMEM((2, 16, 128), jnp.int32),
            pltpu.VMEM((2, U4), jnp.float32),
            pltpu.VMEM((RPT,), jnp.float32),
            pltpu.VMEM_SHARED((NP,), jnp.float32),
            pltpu.SemaphoreType.DMA,
            pltpu.SemaphoreType.DMA,
            pltpu.SemaphoreType.DMA,
            pltpu.SemaphoreType.DMA,
            pltpu.SemaphoreType.DMA,
        ],
    )
    def k4(y_hbm, src_hbm, dst_hbm, out_hbm, sidx, didx, vals, zbuf, acc_sh,
           semi, semg0, semg1, sems0, sems1):
        c = lax.axis_index("c")
        s = lax.axis_index("s")
        semg = (semg0, semg1)
        sems = (sems0, sems1)

        _zero_vmem_1d(zbuf, RPT)
        pltpu.sync_copy(zbuf, acc_sh.at[pl.ds(s * RPT, RPT)])
        plsc.subcore_barrier()

        row0 = s * (UPT4 * 16)

        def load_idx_sync(slot, u):
            r = row0 + u * 16
            h1 = pltpu.async_copy(src_hbm.at[c, pl.ds(r, 16)], sidx.at[slot],
                                  semi)
            h2 = pltpu.async_copy(dst_hbm.at[pl.ds(r, 16)], didx.at[slot],
                                  semi)
            h1.wait()
            h2.wait()

        def fire_g(slot):
            for j in range(16):
                pltpu.async_copy(y_hbm.at[sidx.at[slot, j]],
                                 vals.at[slot, pl.ds(j * 128, 128)],
                                 semg[slot])

        def wait_g(slot):
            for j in range(16):
                pltpu.make_async_copy(y_hbm.at[pl.ds(0, 128)],
                                      vals.at[slot, pl.ds(j * 128, 128)],
                                      semg[slot]).wait()

        def fire_s(slot):
            for j in range(16):
                pltpu.async_copy(vals.at[slot, pl.ds(j * 128, 128)],
                                 acc_sh.at[didx.at[slot, j]], sems[slot],
                                 add=True)

        def wait_s(slot):
            for j in range(16):
                pltpu.make_async_copy(vals.at[slot, pl.ds(j * 128, 128)],
                                      acc_sh.at[pl.ds(0, 128)],
                                      sems[slot]).wait()

        for slot in (0, 1):
            load_idx_sync(slot, slot)
            fire_g(slot)
            wait_g(slot)
            fire_s(slot)

        @pl.loop(0, (UPT4 - 2) // 2)
        def _(p):
            u = 2 + p * 2
            for slot in (0, 1):
                wait_s(slot)
                load_idx_sync(slot, u + slot)
                fire_g(slot)
                wait_g(slot)
                fire_s(slot)

        wait_s(0)
        wait_s(1)

        plsc.subcore_barrier()
        pltpu.sync_copy(acc_sh.at[pl.ds(s * RPT, RPT)],
                        out_hbm.at[pl.ds(c * NP + s * RPT, RPT)])

    return k4(y, src3, dst2d)


def _k1_body(feat_ref, wih_ref, whh_ref, bih_ref, bhh_ref, ws_ref, bs_ref,
             wf_ref, bf_ref, out_ref):
    blk = feat_ref.shape[0]
    wih = wih_ref[...]
    whh = whh_ref[...]
    bias = bih_ref[...] + bhh_ref[...]
    h = jnp.zeros((blk, H), jnp.float32)
    c = jnp.zeros((blk, H), jnp.float32)
    dn = (((1,), (1,)), ((), ()))
    for t in range(T):
        x_t = feat_ref[:, t * F_DYN:(t + 1) * F_DYN]
        gates = (lax.dot_general(x_t, wih, dn, preferred_element_type=jnp.float32)
                 + lax.dot_general(h, whh, dn, preferred_element_type=jnp.float32)
                 + bias)
        gi = jax.nn.sigmoid(gates[:, 0 * H:1 * H])
        gf = jax.nn.sigmoid(gates[:, 1 * H:2 * H])
        gg = jnp.tanh(gates[:, 2 * H:3 * H])
        go = jax.nn.sigmoid(gates[:, 3 * H:4 * H])
        c = gf * c + gi * gg
        h = go * jnp.tanh(c)
    sta = feat_ref[:, T * F_DYN:T * F_DYN + F_STA]
    s_t = jax.nn.relu(
        lax.dot_general(sta, ws_ref[...], dn, preferred_element_type=jnp.float32)
        + bs_ref[...])
    wf = wf_ref[...]
    emb = jax.nn.relu(
        lax.dot_general(h, wf[:, :H], dn, preferred_element_type=jnp.float32)
        + lax.dot_general(s_t, wf[:, H:], dn, preferred_element_type=jnp.float32)
        + bf_ref[...])
    if GP > H:
        emb = jnp.concatenate([emb, jnp.zeros((blk, GP - H), jnp.float32)],
                              axis=1)
    out_ref[...] = emb


def _k1_encode(feat, W_ih, W_hh, b_ih, b_hh, W_s, b_s, W_f, b_f):
    blk = 1024
    grid = (B * NP // blk,)
    return pl.pallas_call(
        _k1_body,
        grid=grid,
        in_specs=[
            pl.BlockSpec((blk, FW), lambda i: (i, 0)),
            pl.BlockSpec((4 * H, F_DYN), lambda i: (0, 0)),
            pl.BlockSpec((4 * H, H), lambda i: (0, 0)),
            pl.BlockSpec((1, 4 * H), lambda i: (0, 0)),
            pl.BlockSpec((1, 4 * H), lambda i: (0, 0)),
            pl.BlockSpec((H, F_STA), lambda i: (0, 0)),
            pl.BlockSpec((1, H), lambda i: (0, 0)),
            pl.BlockSpec((H, 2 * H), lambda i: (0, 0)),
            pl.BlockSpec((1, H), lambda i: (0, 0)),
        ],
        out_specs=pl.BlockSpec((blk, GP), lambda i: (i, 0)),
        out_shape=jax.ShapeDtypeStruct((B * NP, GP), jnp.float32),
    )(feat, W_ih, W_hh, b_ih, b_hh, W_s, b_s, W_f, b_f)


def _k3_body(sum1_ref, x_ref, cnt_ref, wl1_ref, bl1_ref, wr1_ref, wl2_ref,
             wr2_ref, wo_ref, y_ref, z_ref):
    inv = 1.0 / jnp.maximum(cnt_ref[...], 1.0)
    mean = sum1_ref[:, :G] * inv
    dn = (((1,), (1,)), ((), ()))
    x1 = jax.nn.relu(
        lax.dot_general(mean, wl1_ref[...], dn, preferred_element_type=jnp.float32)
        + lax.dot_general(x_ref[:, :G], wr1_ref[...], dn,
                          preferred_element_type=jnp.float32)
        + bl1_ref[...])
    wo = wo_ref[...]
    vl = lax.dot_general(wo, wl2_ref[...], (((1,), (0,)), ((), ())),
                         preferred_element_type=jnp.float32)
    vr = lax.dot_general(wo, wr2_ref[...], (((1,), (0,)), ((), ())),
                         preferred_element_type=jnp.float32)
    y_ref[...] = jnp.sum(x1 * vl, axis=1, keepdims=True)
    z_ref[...] = jnp.sum(x1 * vr, axis=1, keepdims=True)


def _k3_sage1(sum1, embed, cnt_col, W_l1, b_l1, W_r1, W_l2, W_r2, W_o):
    blk = 2048
    grid = (B * NP // blk,)
    return pl.pallas_call(
        _k3_body,
        grid=grid,
        in_specs=[
            pl.BlockSpec((blk, GP), lambda i: (i, 0)),
            pl.BlockSpec((blk, GP), lambda i: (i, 0)),
            pl.BlockSpec((blk, 1), lambda i: (i, 0)),
            pl.BlockSpec((G, G), lambda i: (0, 0)),
            pl.BlockSpec((1, G), lambda i: (0, 0)),
            pl.BlockSpec((G, G), lambda i: (0, 0)),
            pl.BlockSpec((G, G), lambda i: (0, 0)),
            pl.BlockSpec((G, G), lambda i: (0, 0)),
            pl.BlockSpec((1, G), lambda i: (0, 0)),
        ],
        out_specs=[
            pl.BlockSpec((blk, 1), lambda i: (i, 0)),
            pl.BlockSpec((blk, 1), lambda i: (i, 0)),
        ],
        out_shape=[
            jax.ShapeDtypeStruct((B * NP, 1), jnp.float32),
            jax.ShapeDtypeStruct((B * NP, 1), jnp.float32),
        ],
    )(sum1, embed, cnt_col, W_l1, b_l1, W_r1, W_l2, W_r2, W_o)


def _k5_body(s2_ref, cnt_ref, z_ref, bl2_ref, wo_ref, bo_ref, out_ref):
    cnt = cnt_ref[0, :N]
    inv = 1.0 / jnp.maximum(cnt, 1.0)
    c0 = jnp.sum(bl2_ref[...] * wo_ref[...]) + bo_ref[0, 0]
    out_ref[...] = s2_ref[:, :N] * inv[None, :] + z_ref[...] + c0


def _k5_combine(s2, cnt, z, b_l2, W_o, b_o):
    return pl.pallas_call(
        _k5_body,
        grid=(1,),
        in_specs=[
            pl.BlockSpec((B, NP), lambda i: (0, 0)),
            pl.BlockSpec((1, NP), lambda i: (0, 0)),
            pl.BlockSpec((B, N), lambda i: (0, 0)),
            pl.BlockSpec((1, G), lambda i: (0, 0)),
            pl.BlockSpec((1, G), lambda i: (0, 0)),
            pl.BlockSpec((1, 1), lambda i: (0, 0)),
        ],
        out_specs=pl.BlockSpec((B, N), lambda i: (0, 0)),
        out_shape=jax.ShapeDtypeStruct((B, N), jnp.float32),
    )(s2, cnt, z, b_l2, W_o, b_o)


def kernel(dynamic_features, static_features, edge_index, W_ih, W_hh, b_ih,
           b_hh, W_s, b_s, W_f, b_f, W_l1, b_l1, W_r1, W_l2, b_l2, W_r2, W_o,
           b_o):
    dyn = jnp.transpose(dynamic_features, (0, 2, 1, 3)).reshape(B, N, T * F_DYN)
    sta = static_features
    feat = jnp.concatenate([dyn, sta], axis=2)
    feat = jnp.pad(feat, ((0, 0), (0, NP - N), (0, 0))).reshape(B * NP, FW)

    src = edge_index[0]
    dst = edge_index[1]
    npad = EPAD - E
    pad_src = (jnp.arange(npad, dtype=jnp.int32) * 37) % N
    pad_dst = N + (jnp.arange(npad, dtype=jnp.int32) % (NP - N))
    src2d = jnp.concatenate([src, pad_src]).reshape(EPAD // 128, 128)
    dst2d = jnp.concatenate([dst, pad_dst]).reshape(EPAD // 128, 128)
    src3 = jnp.stack([src2d, src2d + NP])

    cnt = _k0_counts(dst2d)
    cnt_col = jnp.concatenate([cnt, cnt]).reshape(B * NP, 1)

    embed = _k1_encode(feat, W_ih, W_hh, b_ih.reshape(1, 4 * H),
                       b_hh.reshape(1, 4 * H), W_s, b_s.reshape(1, H), W_f,
                       b_f.reshape(1, H))

    sum1 = _k2_seg64(embed, src3, dst2d)

    y1, z1 = _k3_sage1(sum1, embed, cnt_col, W_l1, b_l1.reshape(1, G), W_r1,
                       W_l2, W_r2, W_o)

    yf = y1.reshape(B * NP)
    s2 = _k4_seg1(yf, src3, dst2d).reshape(B, NP)

    z = z1.reshape(B, NP)[:, :N]
    pred = _k5_combine(s2, cnt.reshape(1, NP), z, b_l2.reshape(1, G), W_o,
                       b_o.reshape(1, 1))
    return pred
